# Initial kernel scaffold; baseline (speedup 1.0000x reference)
import jax
import jax.numpy as jnp
from jax.experimental import pallas as pl


def kernel(x, edge_index, W1, b1, W2, b2, W3, b3):
    raise NotImplementedError("write your pallas kernel here")


# speedup vs baseline: 7.3164x; 7.3164x over previous
import functools

import jax
import jax.numpy as jnp
from jax import lax
from jax.experimental import pallas as pl
from jax.experimental.pallas import tpu as pltpu
from jax.experimental.pallas import tpu_sc as plsc

_NC = 2
_NS = 16
_NW = _NC * _NS
_C = 80
_NZ = 5


def _sc_mesh():
    return plsc.VectorSubcoreMesh(
        core_axis_name="c", subcore_axis_name="s",
        num_cores=_NC, num_subcores=_NS)


_SC_PARAMS = pltpu.CompilerParams(use_tc_tiling_on_sc=False)


def _make_deg(n_nodes, n_chunks):
    cpt = n_chunks // _NW
    zblk = n_nodes // _NZ

    @functools.partial(
        pl.kernel,
        out_type=jax.ShapeDtypeStruct((_NC, n_nodes), jnp.float32),
        mesh=_sc_mesh(),
        scratch_types=[
            pltpu.VMEM((cpt, _C), jnp.int32),
            pltpu.VMEM((_C,), jnp.float32),
            pltpu.VMEM_SHARED((n_nodes,), jnp.float32),
            pltpu.SemaphoreType.DMA,
        ],
        compiler_params=_SC_PARAMS,
    )
    def deg_kernel(dst_hbm, zeros_hbm, out_hbm, dst_v, ones_v, deg_sh, sem):
        c = lax.axis_index("c")
        s = lax.axis_index("s")
        w = s * _NC + c
        for i in range(_C // 16):
            ones_v[pl.ds(i * 16, 16)] = jnp.ones((16,), jnp.float32)
        pltpu.sync_copy(dst_hbm.at[pl.ds(w * cpt, cpt)], dst_v)

        @pl.when(s < _NZ)
        def _zero():
            pltpu.sync_copy(zeros_hbm.at[pl.ds(s * zblk, zblk)],
                            deg_sh.at[pl.ds(s * zblk, zblk)])

        plsc.subcore_barrier()

        def body(j, carry):
            pltpu.sync_copy(ones_v, deg_sh.at[dst_v.at[j]], add=True)
            return carry

        lax.fori_loop(0, cpt, body, 0)
        plsc.subcore_barrier()

        @pl.when(s < _NZ)
        def _emit():
            pltpu.sync_copy(deg_sh.at[pl.ds(s * zblk, zblk)],
                            out_hbm.at[c, pl.ds(s * zblk, zblk)])

    return deg_kernel


def _make_agg(n_nodes, n_chunks, d):
    cpt = n_chunks // _NW
    rpt = n_nodes // _NS

    @functools.partial(
        pl.kernel,
        out_type=jax.ShapeDtypeStruct((_NC, n_nodes, d), jnp.float32),
        mesh=_sc_mesh(),
        scratch_types=[
            pltpu.VMEM((cpt, _C), jnp.int32),
            pltpu.VMEM((cpt, _C), jnp.int32),
            pltpu.VMEM((_C, d), jnp.float32),
            pltpu.VMEM_SHARED((n_nodes, d), jnp.float32),
            pltpu.SemaphoreType.DMA,
        ],
        compiler_params=_SC_PARAMS,
    )
    def agg_kernel(x_hbm, src_hbm, dst_hbm, zeros_hbm, out_hbm,
                   src_v, dst_v, rows_v, acc_sh, sem):
        c = lax.axis_index("c")
        s = lax.axis_index("s")
        w = s * _NC + c
        pltpu.sync_copy(src_hbm.at[pl.ds(w * cpt, cpt)], src_v)
        pltpu.sync_copy(dst_hbm.at[pl.ds(w * cpt, cpt)], dst_v)
        pltpu.sync_copy(zeros_hbm.at[pl.ds(s * rpt, rpt)],
                        acc_sh.at[pl.ds(s * rpt, rpt)])
        plsc.subcore_barrier()

        def body(j, carry):
            pltpu.async_copy(x_hbm.at[src_v.at[j]], rows_v, sem).wait()
            pltpu.sync_copy(rows_v, acc_sh.at[dst_v.at[j]], add=True)
            return carry

        lax.fori_loop(0, cpt, body, 0)
        plsc.subcore_barrier()
        pltpu.sync_copy(acc_sh.at[pl.ds(s * rpt, rpt)],
                        out_hbm.at[c, pl.ds(s * rpt, rpt)])

    return agg_kernel


def _norm_from_deg(degr):
    dsum = degr[0] + degr[1]
    return jnp.where(dsum > 0, lax.rsqrt(jnp.maximum(dsum, 1.0)), 0.0)


def _scale_kernel(deg_ref, x_ref, o_ref):
    o_ref[...] = x_ref[...] * _norm_from_deg(deg_ref[...])


def _layer1_kernel(deg_ref, p_ref, w_ref, b_ref, o_ref):
    norm = _norm_from_deg(deg_ref[...])
    agg = (p_ref[0] + p_ref[1]) * norm
    h = jnp.dot(agg, w_ref[...], preferred_element_type=jnp.float32) + b_ref[...]
    o_ref[...] = jnp.maximum(h, 0.0) * norm


def _layer2_kernel(deg_ref, p_ref, w2_ref, b2_ref, w3_ref, b3_ref,
                   o_ref, acc_ref, *, nblocks, n_nodes):
    i = pl.program_id(0)
    norm = _norm_from_deg(deg_ref[...])
    agg = (p_ref[0] + p_ref[1]) * norm
    h = jnp.dot(agg, w2_ref[...], preferred_element_type=jnp.float32) + b2_ref[...]
    h = jnp.maximum(h, 0.0)
    csum = jnp.sum(h, axis=0, keepdims=True)

    @pl.when(i == 0)
    def _init():
        acc_ref[...] = jnp.zeros_like(acc_ref)

    acc_ref[...] += csum

    @pl.when(i == nblocks - 1)
    def _final():
        hg = acc_ref[...] * (1.0 / n_nodes)
        o_ref[...] = (jnp.dot(hg, w3_ref[...], preferred_element_type=jnp.float32)
                      + b3_ref[...])


def kernel(x, edge_index, W1, b1, W2, b2, W3, b3):
    n, din = x.shape
    e = edge_index.shape[1]
    hid = W1.shape[1]
    out2 = W2.shape[1]
    ncls = W3.shape[1]
    nch = e // _C

    src2d = edge_index[0].reshape(nch, _C)
    dst2d = edge_index[1].reshape(nch, _C)
    zeros1 = jnp.zeros((n,), jnp.float32)
    zerosf = jnp.zeros((n, din), jnp.float32)

    deg = _make_deg(n, nch)(dst2d, zeros1)
    degcol = deg.reshape(_NC, n, 1)

    BLK = 1000
    gridn = n // BLK

    xs = pl.pallas_call(
        _scale_kernel,
        grid=(gridn,),
        in_specs=[
            pl.BlockSpec((_NC, BLK, 1), lambda i: (0, i, 0)),
            pl.BlockSpec((BLK, din), lambda i: (i, 0)),
        ],
        out_specs=pl.BlockSpec((BLK, din), lambda i: (i, 0)),
        out_shape=jax.ShapeDtypeStruct((n, din), jnp.float32),
    )(degcol, x)

    p1 = _make_agg(n, nch, din)(xs, src2d, dst2d, zerosf)

    h1s = pl.pallas_call(
        _layer1_kernel,
        grid=(gridn,),
        in_specs=[
            pl.BlockSpec((_NC, BLK, 1), lambda i: (0, i, 0)),
            pl.BlockSpec((_NC, BLK, din), lambda i: (0, i, 0)),
            pl.BlockSpec((din, hid), lambda i: (0, 0)),
            pl.BlockSpec((1, hid), lambda i: (0, 0)),
        ],
        out_specs=pl.BlockSpec((BLK, hid), lambda i: (i, 0)),
        out_shape=jax.ShapeDtypeStruct((n, hid), jnp.float32),
    )(degcol, p1, W1, b1.reshape(1, hid))

    p2 = _make_agg(n, nch, hid)(h1s, src2d, dst2d, zerosf)

    y = pl.pallas_call(
        functools.partial(_layer2_kernel, nblocks=gridn, n_nodes=n),
        grid=(gridn,),
        in_specs=[
            pl.BlockSpec((_NC, BLK, 1), lambda i: (0, i, 0)),
            pl.BlockSpec((_NC, BLK, hid), lambda i: (0, i, 0)),
            pl.BlockSpec((hid, out2), lambda i: (0, 0)),
            pl.BlockSpec((1, out2), lambda i: (0, 0)),
            pl.BlockSpec((out2, ncls), lambda i: (0, 0)),
            pl.BlockSpec((1, ncls), lambda i: (0, 0)),
        ],
        out_specs=pl.BlockSpec((1, ncls), lambda i: (0, 0)),
        out_shape=jax.ShapeDtypeStruct((1, ncls), jnp.float32),
        scratch_shapes=[pltpu.VMEM((1, out2), jnp.float32)],
    )(degcol, p2, W2, b2.reshape(1, out2), W3, b3.reshape(1, ncls))

    return y


# speedup vs baseline: 11.6697x; 1.5950x over previous
import functools

import jax
import jax.numpy as jnp
from jax import lax
from jax.experimental import pallas as pl
from jax.experimental.pallas import tpu as pltpu
from jax.experimental.pallas import tpu_sc as plsc

_NC = 2
_NS = 16
_NW = _NC * _NS
_C = 100
_K = 2
_NZ = 5


def _sc_mesh():
    return plsc.VectorSubcoreMesh(
        core_axis_name="c", subcore_axis_name="s",
        num_cores=_NC, num_subcores=_NS)


_SC_PARAMS = pltpu.CompilerParams(use_tc_tiling_on_sc=False)


def _make_deg(n_nodes, n_chunks):
    cpt = n_chunks // _NW
    zblk = n_nodes // _NZ

    @functools.partial(
        pl.kernel,
        out_type=jax.ShapeDtypeStruct((_NC, n_nodes), jnp.float32),
        mesh=_sc_mesh(),
        scratch_types=[
            pltpu.VMEM((cpt, _C), jnp.int32),
            pltpu.VMEM((128,), jnp.float32),
            pltpu.VMEM_SHARED((n_nodes,), jnp.float32),
            pltpu.SemaphoreType.DMA,
        ],
        compiler_params=_SC_PARAMS,
    )
    def deg_kernel(dst_hbm, zeros_hbm, out_hbm, dst_v, ones_v, deg_sh, sem):
        c = lax.axis_index("c")
        s = lax.axis_index("s")
        w = s * _NC + c
        for i in range(8):
            ones_v[pl.ds(i * 16, 16)] = jnp.ones((16,), jnp.float32)
        pltpu.sync_copy(dst_hbm.at[pl.ds(w * cpt, cpt)], dst_v)

        @pl.when(s < _NZ)
        def _zero():
            pltpu.sync_copy(zeros_hbm.at[pl.ds(s * zblk, zblk)],
                            deg_sh.at[pl.ds(s * zblk, zblk)])

        plsc.subcore_barrier()

        def body(j, carry):
            pltpu.sync_copy(ones_v.at[pl.ds(0, _C)], deg_sh.at[dst_v.at[j]],
                            add=True)
            return carry

        lax.fori_loop(0, cpt, body, 0)
        plsc.subcore_barrier()

        @pl.when(s < _NZ)
        def _emit():
            pltpu.sync_copy(deg_sh.at[pl.ds(s * zblk, zblk)],
                            out_hbm.at[c, pl.ds(s * zblk, zblk)])

    return deg_kernel


def _make_agg(n_nodes, n_chunks, d):
    cpt = n_chunks // _NW
    rpt = n_nodes // _NS

    @functools.partial(
        pl.kernel,
        out_type=jax.ShapeDtypeStruct((_NC, n_nodes, d), jnp.float32),
        mesh=_sc_mesh(),
        scratch_types=[
            pltpu.VMEM_SHARED((n_nodes, d), jnp.float32),
            pltpu.VMEM((cpt, _C), jnp.int32),
            pltpu.VMEM((cpt, _C), jnp.int32),
            [pltpu.VMEM((_C, d), jnp.float32) for _ in range(_K)],
            pltpu.SemaphoreType.DMA((_K,)),
            pltpu.SemaphoreType.DMA((_K,)),
        ],
        compiler_params=_SC_PARAMS,
    )
    def agg_kernel(x_hbm, src_hbm, dst_hbm, zeros_hbm, out_hbm,
                   acc_sh, src_v, dst_v, rows, gsem, ssem):
        c = lax.axis_index("c")
        s = lax.axis_index("s")
        w = s * _NC + c
        pltpu.sync_copy(src_hbm.at[pl.ds(w * cpt, cpt)], src_v)
        pltpu.sync_copy(dst_hbm.at[pl.ds(w * cpt, cpt)], dst_v)
        pltpu.sync_copy(zeros_hbm.at[pl.ds(s * rpt, rpt)],
                        acc_sh.at[pl.ds(s * rpt, rpt)])
        plsc.subcore_barrier()

        for b in range(_K):
            pltpu.async_copy(x_hbm.at[src_v.at[b]], rows[b], gsem.at[b])

        def round_body(r, carry):
            for b in range(_K):
                j = r * _K + b
                pltpu.make_async_copy(x_hbm.at[src_v.at[j]], rows[b],
                                      gsem.at[b]).wait()
                pltpu.async_copy(rows[b], acc_sh.at[dst_v.at[j]], ssem.at[b],
                                 add=True)

                @pl.when(j + _K < cpt)
                def _refill():
                    pltpu.make_async_copy(rows[b], acc_sh.at[dst_v.at[j]],
                                          ssem.at[b]).wait()
                    pltpu.async_copy(x_hbm.at[src_v.at[j + _K]], rows[b],
                                    gsem.at[b])
            return carry

        lax.fori_loop(0, cpt // _K, round_body, 0)
        for b in range(_K):
            pltpu.make_async_copy(rows[b], acc_sh.at[dst_v.at[cpt - _K + b]],
                                  ssem.at[b]).wait()
        plsc.subcore_barrier()
        pltpu.sync_copy(acc_sh.at[pl.ds(s * rpt, rpt)],
                        out_hbm.at[c, pl.ds(s * rpt, rpt)])

    return agg_kernel


def _norm_from_deg(degr):
    dsum = degr[0] + degr[1]
    return jnp.where(dsum > 0, lax.rsqrt(jnp.maximum(dsum, 1.0)), 0.0)


def _scale_kernel(deg_ref, x_ref, o_ref):
    o_ref[...] = x_ref[...] * _norm_from_deg(deg_ref[...])


def _layer1_kernel(deg_ref, p_ref, w_ref, b_ref, o_ref):
    norm = _norm_from_deg(deg_ref[...])
    agg = (p_ref[0] + p_ref[1]) * norm
    h = jnp.dot(agg, w_ref[...], preferred_element_type=jnp.float32) + b_ref[...]
    o_ref[...] = jnp.maximum(h, 0.0) * norm


def _layer2_kernel(deg_ref, p_ref, w2_ref, b2_ref, w3_ref, b3_ref,
                   o_ref, acc_ref, *, nblocks, n_nodes):
    i = pl.program_id(0)
    norm = _norm_from_deg(deg_ref[...])
    agg = (p_ref[0] + p_ref[1]) * norm
    h = jnp.dot(agg, w2_ref[...], preferred_element_type=jnp.float32) + b2_ref[...]
    h = jnp.maximum(h, 0.0)
    csum = jnp.sum(h, axis=0, keepdims=True)

    @pl.when(i == 0)
    def _init():
        acc_ref[...] = jnp.zeros_like(acc_ref)

    acc_ref[...] += csum

    @pl.when(i == nblocks - 1)
    def _final():
        hg = acc_ref[...] * (1.0 / n_nodes)
        o_ref[...] = (jnp.dot(hg, w3_ref[...], preferred_element_type=jnp.float32)
                      + b3_ref[...])


def kernel(x, edge_index, W1, b1, W2, b2, W3, b3):
    n, din = x.shape
    e = edge_index.shape[1]
    hid = W1.shape[1]
    out2 = W2.shape[1]
    ncls = W3.shape[1]
    nch = e // _C

    src2d = edge_index[0].reshape(nch, _C)
    dst2d = edge_index[1].reshape(nch, _C)
    zeros1 = jnp.zeros((n,), jnp.float32)
    zerosf = jnp.zeros((n, din), jnp.float32)

    deg = _make_deg(n, nch)(dst2d, zeros1)
    degcol = deg.reshape(_NC, n, 1)

    BLK = 1000
    gridn = n // BLK

    xs = pl.pallas_call(
        _scale_kernel,
        grid=(gridn,),
        in_specs=[
            pl.BlockSpec((_NC, BLK, 1), lambda i: (0, i, 0)),
            pl.BlockSpec((BLK, din), lambda i: (i, 0)),
        ],
        out_specs=pl.BlockSpec((BLK, din), lambda i: (i, 0)),
        out_shape=jax.ShapeDtypeStruct((n, din), jnp.float32),
    )(degcol, x)

    p1 = _make_agg(n, nch, din)(xs, src2d, dst2d, zerosf)

    h1s = pl.pallas_call(
        _layer1_kernel,
        grid=(gridn,),
        in_specs=[
            pl.BlockSpec((_NC, BLK, 1), lambda i: (0, i, 0)),
            pl.BlockSpec((_NC, BLK, din), lambda i: (0, i, 0)),
            pl.BlockSpec((din, hid), lambda i: (0, 0)),
            pl.BlockSpec((1, hid), lambda i: (0, 0)),
        ],
        out_specs=pl.BlockSpec((BLK, hid), lambda i: (i, 0)),
        out_shape=jax.ShapeDtypeStruct((n, hid), jnp.float32),
    )(degcol, p1, W1, b1.reshape(1, hid))

    p2 = _make_agg(n, nch, hid)(h1s, src2d, dst2d, zerosf)

    y = pl.pallas_call(
        functools.partial(_layer2_kernel, nblocks=gridn, n_nodes=n),
        grid=(gridn,),
        in_specs=[
            pl.BlockSpec((_NC, BLK, 1), lambda i: (0, i, 0)),
            pl.BlockSpec((_NC, BLK, hid), lambda i: (0, i, 0)),
            pl.BlockSpec((hid, out2), lambda i: (0, 0)),
            pl.BlockSpec((1, out2), lambda i: (0, 0)),
            pl.BlockSpec((out2, ncls), lambda i: (0, 0)),
            pl.BlockSpec((1, ncls), lambda i: (0, 0)),
        ],
        out_specs=pl.BlockSpec((1, ncls), lambda i: (0, 0)),
        out_shape=jax.ShapeDtypeStruct((1, ncls), jnp.float32),
        scratch_shapes=[pltpu.VMEM((1, out2), jnp.float32)],
    )(degcol, p2, W2, b2.reshape(1, out2), W3, b3.reshape(1, ncls))

    return y
